# baseline (speedup 1.0000x reference)
import jax
import jax.numpy as jnp
from jax.experimental import pallas as pl
from jax.experimental.pallas import tpu as pltpu

N = 4096
DIM = 256
CLASSES = 100
CPAD = 128
STRIP = 512
NSTRIPS = N // STRIP
SIGMA = 1.0
BIG = 3.0e38
BIGI = 2 ** 30
PACK = 128


def _dgmm_kernel(x_ref, xt_ref, yrow_ref, pcol_ref, pself_ref, ybf_ref,
                 out_ref, mut_ref, mm_ref, cnt_ref, xxc_ref):
    step = pl.program_id(0)

    @pl.when(step == 0)
    def _prologue():
        yrow = yrow_ref[...]
        cls_iota = jax.lax.broadcasted_iota(jnp.int32, (1, CPAD), 1)
        onehot = (yrow == cls_iota).astype(jnp.float32)
        counts = jnp.sum(onehot, axis=0, keepdims=True)
        sums_t = jax.lax.dot_general(
            xt_ref[...], onehot, (((1,), (0,)), ((), ())),
            preferred_element_type=jnp.float32)
        mut = sums_t / jnp.maximum(counts, 1.0)
        mut_ref[...] = mut
        mm_ref[...] = jnp.sum(mut * mut, axis=0, keepdims=True)
        cnt_ref[...] = counts
        xt = xt_ref[...]
        xxc_ref[...] = jnp.sum(xt * xt, axis=0, keepdims=True)
        out_ref[...] = jnp.zeros_like(out_ref)

    r0 = step * STRIP
    xs = x_ref[pl.ds(r0, STRIP), :]
    yr = yrow_ref[pl.ds(r0, STRIP), :]

    xx_row = jnp.sum(xs * xs, axis=1, keepdims=True)
    xmu = jax.lax.dot_general(xs, mut_ref[...], (((1,), (0,)), ((), ())),
                              preferred_element_type=jnp.float32)
    d2 = xx_row - 2.0 * xmu + mm_ref[...]
    pi = jnp.exp(d2 * (-0.5 / (SIGMA * SIGMA)))
    pi = jnp.where(cnt_ref[...] > 0.0, pi, 0.0)
    s = jnp.sum(pi, axis=1, keepdims=True)
    pi = pi / (s + 1e-15)
    pi = jnp.clip(pi, 0.0, 1.0)
    cls_iota = jax.lax.broadcasted_iota(jnp.int32, (1, CPAD), 1)
    yh = (yr == cls_iota).astype(jnp.float32)
    gm = jnp.sum((pi - yh) ** 2, keepdims=True)

    prod = jax.lax.dot_general(xs + xs, xt_ref[...], (((1,), (0,)), ((), ())),
                               preferred_element_type=jnp.float32)
    t = xxc_ref[...] - prod
    pcol = pcol_ref[...]
    t = jnp.where(pcol == pself_ref[pl.ds(r0, STRIP), :], BIG, t)

    ybf = ybf_ref[...]
    labs = []
    for r in range(3):
        m = jnp.min(t, axis=1, keepdims=True)
        at_min = t == m
        mask_bf = jnp.where(at_min, 1.0, 0.0).astype(jnp.bfloat16)
        lab_f = jax.lax.dot_general(mask_bf, ybf, (((1,), (0,)), ((), ())),
                                    preferred_element_type=jnp.float32)
        labs.append(jnp.minimum(lab_f.astype(jnp.int32), CLASSES - 1))
        if r < 2:
            t = jnp.where(at_min, BIG, t)

    l1, l2, l3 = labs
    a = jnp.minimum(l1, jnp.minimum(l2, l3))
    c = jnp.maximum(l1, jnp.maximum(l2, l3))
    b = l1 + l2 + l3 - a - c
    mode = jnp.where(b == c, b, a)
    diff = (mode - yr).astype(jnp.float32)
    knn = jnp.sum(diff * diff, keepdims=True)

    out_ref[...] += (gm + 0.01 * knn) * (1.0 / N)


@jax.jit
def kernel(x, y):
    xt = x.T
    yrow = y.reshape(N, 1)
    idx = jnp.arange(N, dtype=jnp.int32)
    packed = idx * PACK + y
    pcol = packed.reshape(1, N)
    pself = packed.reshape(N, 1)
    ybf = y.astype(jnp.bfloat16).reshape(N, 1)
    out = pl.pallas_call(
        _dgmm_kernel,
        grid=(NSTRIPS,),
        out_shape=jax.ShapeDtypeStruct((1, 1), jnp.float32),
        scratch_shapes=[
            pltpu.VMEM((DIM, CPAD), jnp.float32),
            pltpu.VMEM((1, CPAD), jnp.float32),
            pltpu.VMEM((1, CPAD), jnp.float32),
            pltpu.VMEM((1, N), jnp.float32),
        ],
    )(x, xt, yrow, pcol, pself, ybf)
    return out[0, 0]


# speedup vs baseline: 1.2878x; 1.2878x over previous
import jax
import jax.numpy as jnp
from jax.experimental import pallas as pl
from jax.experimental.pallas import tpu as pltpu

N = 4096
DIM = 256
CLASSES = 100
CPAD = 128
STRIP = 1024
NSTRIPS = N // STRIP
SIGMA = 1.0
BIG = 3.0e38
BIGI = 2 ** 30
PACK = 128


def _dgmm_kernel(x_ref, xt_ref, yrow_ref, pcol_ref, pself_ref,
                 out_ref, mut_ref, mm_ref, cnt_ref, xxc_ref):
    step = pl.program_id(0)

    @pl.when(step == 0)
    def _prologue():
        yrow = yrow_ref[...]
        cls_iota = jax.lax.broadcasted_iota(jnp.int32, (1, CPAD), 1)
        onehot = (yrow == cls_iota).astype(jnp.float32)
        counts = jnp.sum(onehot, axis=0, keepdims=True)
        sums_t = jax.lax.dot_general(
            xt_ref[...], onehot, (((1,), (0,)), ((), ())),
            preferred_element_type=jnp.float32)
        mut = sums_t / jnp.maximum(counts, 1.0)
        mut_ref[...] = mut
        mm_ref[...] = jnp.sum(mut * mut, axis=0, keepdims=True)
        cnt_ref[...] = counts
        xt = xt_ref[...]
        xxc_ref[...] = jnp.sum(xt * xt, axis=0, keepdims=True)
        out_ref[...] = jnp.zeros_like(out_ref)

    r0 = step * STRIP
    xs = x_ref[pl.ds(r0, STRIP), :]
    yr = yrow_ref[pl.ds(r0, STRIP), :]

    xx_row = jnp.sum(xs * xs, axis=1, keepdims=True)
    xmu = jax.lax.dot_general(xs, mut_ref[...], (((1,), (0,)), ((), ())),
                              preferred_element_type=jnp.float32)
    d2 = xx_row - 2.0 * xmu + mm_ref[...]
    pi = jnp.exp(d2 * (-0.5 / (SIGMA * SIGMA)))
    pi = jnp.where(cnt_ref[...] > 0.0, pi, 0.0)
    s = jnp.sum(pi, axis=1, keepdims=True)
    pi = pi / (s + 1e-15)
    pi = jnp.clip(pi, 0.0, 1.0)
    cls_iota = jax.lax.broadcasted_iota(jnp.int32, (1, CPAD), 1)
    yh = (yr == cls_iota).astype(jnp.float32)
    gm = jnp.sum((pi - yh) ** 2, keepdims=True)

    prod = jax.lax.dot_general(xs + xs, xt_ref[...], (((1,), (0,)), ((), ())),
                               preferred_element_type=jnp.float32)
    t = xxc_ref[...] - prod
    pcol = pcol_ref[...]
    t = jnp.where(pcol == pself_ref[pl.ds(r0, STRIP), :], BIG, t)

    labs = []
    for r in range(3):
        m = jnp.min(t, axis=1, keepdims=True)
        at_min = t == m
        pk = jnp.min(jnp.where(at_min, pcol, BIGI),
                     axis=1, keepdims=True)
        labs.append(pk & (PACK - 1))
        if r < 2:
            t = jnp.where(at_min, BIG, t)

    l1, l2, l3 = labs
    a = jnp.minimum(l1, jnp.minimum(l2, l3))
    c = jnp.maximum(l1, jnp.maximum(l2, l3))
    b = l1 + l2 + l3 - a - c
    mode = jnp.where(b == c, b, a)
    diff = (mode - yr).astype(jnp.float32)
    knn = jnp.sum(diff * diff, keepdims=True)

    out_ref[...] += (gm + 0.01 * knn) * (1.0 / N)


@jax.jit
def kernel(x, y):
    xt = x.T
    yrow = y.reshape(N, 1)
    idx = jnp.arange(N, dtype=jnp.int32)
    packed = idx * PACK + y
    pcol = packed.reshape(1, N)
    pself = packed.reshape(N, 1)
    out = pl.pallas_call(
        _dgmm_kernel,
        grid=(NSTRIPS,),
        out_shape=jax.ShapeDtypeStruct((1, 1), jnp.float32),
        scratch_shapes=[
            pltpu.VMEM((DIM, CPAD), jnp.float32),
            pltpu.VMEM((1, CPAD), jnp.float32),
            pltpu.VMEM((1, CPAD), jnp.float32),
            pltpu.VMEM((1, N), jnp.float32),
        ],
    )(x, xt, yrow, pcol, pself)
    return out[0, 0]


# speedup vs baseline: 1.3923x; 1.0811x over previous
import jax
import jax.numpy as jnp
from jax.experimental import pallas as pl
from jax.experimental.pallas import tpu as pltpu

N = 4096
DIM = 256
CLASSES = 100
CPAD = 128
STRIP = 1024
NSTRIPS = N // STRIP
SIGMA = 1.0
BIG = 3.0e38
BIGI = 2 ** 30
PACK = 128


def _dgmm_kernel(x_ref, yrow_ref, pcol_ref, pself_ref,
                 out_ref, mut_ref, mm_ref, cnt_ref, xxc_ref):
    step = pl.program_id(0)

    @pl.when(step == 0)
    def _prologue():
        yrow = yrow_ref[...]
        cls_iota = jax.lax.broadcasted_iota(jnp.int32, (1, CPAD), 1)
        onehot = (yrow == cls_iota).astype(jnp.float32)
        counts = jnp.sum(onehot, axis=0, keepdims=True)
        x = x_ref[...]
        sums_t = jax.lax.dot_general(
            x, onehot, (((0,), (0,)), ((), ())),
            preferred_element_type=jnp.float32)
        mut = sums_t / jnp.maximum(counts, 1.0)
        mut_ref[...] = mut
        mm_ref[...] = jnp.sum(mut * mut, axis=0, keepdims=True)
        cnt_ref[...] = counts
        xsq = x * x
        ones_row = jnp.ones((1, DIM), jnp.float32)
        xxc_ref[...] = jax.lax.dot_general(
            ones_row, xsq, (((1,), (1,)), ((), ())),
            preferred_element_type=jnp.float32)
        out_ref[...] = jnp.zeros_like(out_ref)

    r0 = step * STRIP
    xs = x_ref[pl.ds(r0, STRIP), :]
    yr = yrow_ref[pl.ds(r0, STRIP), :]

    xx_row = jnp.sum(xs * xs, axis=1, keepdims=True)
    xmu = jax.lax.dot_general(xs, mut_ref[...], (((1,), (0,)), ((), ())),
                              preferred_element_type=jnp.float32)
    d2 = xx_row - 2.0 * xmu + mm_ref[...]
    pi = jnp.exp(d2 * (-0.5 / (SIGMA * SIGMA)))
    pi = jnp.where(cnt_ref[...] > 0.0, pi, 0.0)
    s = jnp.sum(pi, axis=1, keepdims=True)
    pi = pi / (s + 1e-15)
    pi = jnp.clip(pi, 0.0, 1.0)
    cls_iota = jax.lax.broadcasted_iota(jnp.int32, (1, CPAD), 1)
    yh = (yr == cls_iota).astype(jnp.float32)
    gm = jnp.sum((pi - yh) ** 2, keepdims=True)

    prod = jax.lax.dot_general(xs + xs, x_ref[...], (((1,), (1,)), ((), ())),
                               preferred_element_type=jnp.float32)
    t = xxc_ref[...] - prod
    pcol = pcol_ref[...]
    t = jnp.where(pcol == pself_ref[pl.ds(r0, STRIP), :], BIG, t)

    labs = []
    for r in range(3):
        m = jnp.min(t, axis=1, keepdims=True)
        at_min = t == m
        pk = jnp.min(jnp.where(at_min, pcol, BIGI),
                     axis=1, keepdims=True)
        labs.append(pk & (PACK - 1))
        if r < 2:
            t = jnp.where(at_min, BIG, t)

    l1, l2, l3 = labs
    a = jnp.minimum(l1, jnp.minimum(l2, l3))
    c = jnp.maximum(l1, jnp.maximum(l2, l3))
    b = l1 + l2 + l3 - a - c
    mode = jnp.where(b == c, b, a)
    diff = (mode - yr).astype(jnp.float32)
    knn = jnp.sum(diff * diff, keepdims=True)

    out_ref[...] += (gm + 0.01 * knn) * (1.0 / N)


@jax.jit
def kernel(x, y):
    yrow = y.reshape(N, 1)
    idx = jnp.arange(N, dtype=jnp.int32)
    packed = idx * PACK + y
    pcol = packed.reshape(1, N)
    pself = packed.reshape(N, 1)
    out = pl.pallas_call(
        _dgmm_kernel,
        grid=(NSTRIPS,),
        out_shape=jax.ShapeDtypeStruct((1, 1), jnp.float32),
        scratch_shapes=[
            pltpu.VMEM((DIM, CPAD), jnp.float32),
            pltpu.VMEM((1, CPAD), jnp.float32),
            pltpu.VMEM((1, CPAD), jnp.float32),
            pltpu.VMEM((1, N), jnp.float32),
        ],
    )(x, yrow, pcol, pself)
    return out[0, 0]


# speedup vs baseline: 1.5309x; 1.0996x over previous
import jax
import jax.numpy as jnp
from jax.experimental import pallas as pl
from jax.experimental.pallas import tpu as pltpu

N = 4096
DIM = 256
CLASSES = 100
CPAD = 128
STRIP = 1024
NSTRIPS = N // STRIP
SIGMA = 1.0
BIG = 3.0e38
BIGI = 2 ** 30
PACK = 128


def _dgmm_kernel(x_ref, yrow_ref, pcol_ref, pself_ref,
                 out_ref, mut_ref, mm_ref, cnt_ref, xxc_ref):
    step = pl.program_id(0)

    @pl.when(step == 0)
    def _prologue():
        yrow = yrow_ref[...]
        cls_iota = jax.lax.broadcasted_iota(jnp.int32, (1, CPAD), 1)
        onehot = (yrow == cls_iota).astype(jnp.float32)
        counts = jnp.sum(onehot, axis=0, keepdims=True)
        x = x_ref[...]
        sums_t = jax.lax.dot_general(
            x, onehot, (((0,), (0,)), ((), ())),
            preferred_element_type=jnp.float32)
        mut = sums_t / jnp.maximum(counts, 1.0)
        mut_ref[...] = mut
        mm_ref[...] = jnp.sum(mut * mut, axis=0, keepdims=True)
        cnt_ref[...] = counts
        xsq = x * x
        ones_row = jnp.ones((1, DIM), jnp.float32)
        xxc_ref[...] = jax.lax.dot_general(
            ones_row, xsq, (((1,), (1,)), ((), ())),
            preferred_element_type=jnp.float32)
        out_ref[...] = jnp.zeros_like(out_ref)

    r0 = step * STRIP
    xs = x_ref[pl.ds(r0, STRIP), :]
    yr = yrow_ref[pl.ds(r0, STRIP), :]

    xx_row = jnp.sum(xs * xs, axis=1, keepdims=True)
    xmu = jax.lax.dot_general(xs, mut_ref[...], (((1,), (0,)), ((), ())),
                              preferred_element_type=jnp.float32)
    d2 = xx_row - 2.0 * xmu + mm_ref[...]
    pi = jnp.exp(d2 * (-0.5 / (SIGMA * SIGMA)))
    pi = jnp.where(cnt_ref[...] > 0.0, pi, 0.0)
    s = jnp.sum(pi, axis=1, keepdims=True)
    pi = pi / (s + 1e-15)
    pi = jnp.clip(pi, 0.0, 1.0)
    cls_iota = jax.lax.broadcasted_iota(jnp.int32, (1, CPAD), 1)
    yh = (yr == cls_iota).astype(jnp.float32)
    gm = jnp.sum((pi - yh) ** 2, keepdims=True)

    prod = jax.lax.dot_general(xs + xs, x_ref[...], (((1,), (1,)), ((), ())),
                               preferred_element_type=jnp.float32)
    t = xxc_ref[...] - prod
    pcol = pcol_ref[...]
    t = jnp.where(pcol == pself_ref[pl.ds(r0, STRIP), :], BIG, t)

    labs = []
    for r in range(3):
        m = jnp.min(t, axis=1, keepdims=True)
        at_min = t == m
        pkf = jnp.min(jnp.where(at_min, pcol, BIG),
                      axis=1, keepdims=True)
        pk = pkf.astype(jnp.int32)
        labs.append(pk & (PACK - 1))
        if r < 2:
            t = jnp.where(at_min, BIG, t)

    l1, l2, l3 = labs
    a = jnp.minimum(l1, jnp.minimum(l2, l3))
    c = jnp.maximum(l1, jnp.maximum(l2, l3))
    b = l1 + l2 + l3 - a - c
    mode = jnp.where(b == c, b, a)
    diff = (mode - yr).astype(jnp.float32)
    knn = jnp.sum(diff * diff, keepdims=True)

    out_ref[...] += (gm + 0.01 * knn) * (1.0 / N)


@jax.jit
def kernel(x, y):
    yrow = y.reshape(N, 1)
    idx = jnp.arange(N, dtype=jnp.int32)
    packed = (idx * PACK + y).astype(jnp.float32)
    pcol = packed.reshape(1, N)
    pself = packed.reshape(N, 1)
    out = pl.pallas_call(
        _dgmm_kernel,
        grid=(NSTRIPS,),
        out_shape=jax.ShapeDtypeStruct((1, 1), jnp.float32),
        scratch_shapes=[
            pltpu.VMEM((DIM, CPAD), jnp.float32),
            pltpu.VMEM((1, CPAD), jnp.float32),
            pltpu.VMEM((1, CPAD), jnp.float32),
            pltpu.VMEM((1, N), jnp.float32),
        ],
    )(x, yrow, pcol, pself)
    return out[0, 0]


# speedup vs baseline: 1.5741x; 1.0282x over previous
import jax
import jax.numpy as jnp
from jax.experimental import pallas as pl
from jax.experimental.pallas import tpu as pltpu

N = 4096
DIM = 256
CLASSES = 100
CPAD = 128
STRIP = 2048
NSTRIPS = N // STRIP
SIGMA = 1.0
BIG = 3.0e38
BIGI = 2 ** 30
PACK = 128


def _dgmm_kernel(x_ref, yrow_ref, pcol_ref, pself_ref,
                 out_ref, mut_ref, mm_ref, cnt_ref, xxc_ref):
    step = pl.program_id(0)

    @pl.when(step == 0)
    def _prologue():
        yrow = yrow_ref[...]
        cls_iota = jax.lax.broadcasted_iota(jnp.int32, (1, CPAD), 1)
        onehot = (yrow == cls_iota).astype(jnp.float32)
        counts = jnp.sum(onehot, axis=0, keepdims=True)
        x = x_ref[...]
        sums_t = jax.lax.dot_general(
            x, onehot, (((0,), (0,)), ((), ())),
            preferred_element_type=jnp.float32)
        mut = sums_t / jnp.maximum(counts, 1.0)
        mut_ref[...] = mut
        mm_ref[...] = jnp.sum(mut * mut, axis=0, keepdims=True)
        cnt_ref[...] = counts
        xsq = x * x
        ones_row = jnp.ones((1, DIM), jnp.float32)
        xxc_ref[...] = jax.lax.dot_general(
            ones_row, xsq, (((1,), (1,)), ((), ())),
            preferred_element_type=jnp.float32)
        out_ref[...] = jnp.zeros_like(out_ref)

    r0 = step * STRIP
    xs = x_ref[pl.ds(r0, STRIP), :]
    yr = yrow_ref[pl.ds(r0, STRIP), :]

    xx_row = jnp.sum(xs * xs, axis=1, keepdims=True)
    xmu = jax.lax.dot_general(xs, mut_ref[...], (((1,), (0,)), ((), ())),
                              preferred_element_type=jnp.float32)
    d2 = xx_row - 2.0 * xmu + mm_ref[...]
    pi = jnp.exp(d2 * (-0.5 / (SIGMA * SIGMA)))
    pi = jnp.where(cnt_ref[...] > 0.0, pi, 0.0)
    s = jnp.sum(pi, axis=1, keepdims=True)
    pi = pi / (s + 1e-15)
    pi = jnp.clip(pi, 0.0, 1.0)
    cls_iota = jax.lax.broadcasted_iota(jnp.int32, (1, CPAD), 1)
    yh = (yr == cls_iota).astype(jnp.float32)
    gm = jnp.sum((pi - yh) ** 2, keepdims=True)

    prod = jax.lax.dot_general(xs + xs, x_ref[...], (((1,), (1,)), ((), ())),
                               preferred_element_type=jnp.float32)
    t = xxc_ref[...] - prod
    pcol = pcol_ref[...]
    t = jnp.where(pcol == pself_ref[pl.ds(r0, STRIP), :], BIG, t)

    labs = []
    for r in range(3):
        m = jnp.min(t, axis=1, keepdims=True)
        at_min = t == m
        pkf = jnp.min(jnp.where(at_min, pcol, BIG),
                      axis=1, keepdims=True)
        pk = pkf.astype(jnp.int32)
        labs.append(pk & (PACK - 1))
        if r < 2:
            t = jnp.where(at_min, BIG, t)

    l1, l2, l3 = labs
    a = jnp.minimum(l1, jnp.minimum(l2, l3))
    c = jnp.maximum(l1, jnp.maximum(l2, l3))
    b = l1 + l2 + l3 - a - c
    mode = jnp.where(b == c, b, a)
    diff = (mode - yr).astype(jnp.float32)
    knn = jnp.sum(diff * diff, keepdims=True)

    out_ref[...] += (gm + 0.01 * knn) * (1.0 / N)


@jax.jit
def kernel(x, y):
    yrow = y.reshape(N, 1)
    idx = jnp.arange(N, dtype=jnp.int32)
    packed = (idx * PACK + y).astype(jnp.float32)
    pcol = packed.reshape(1, N)
    pself = packed.reshape(N, 1)
    out = pl.pallas_call(
        _dgmm_kernel,
        grid=(NSTRIPS,),
        out_shape=jax.ShapeDtypeStruct((1, 1), jnp.float32),
        scratch_shapes=[
            pltpu.VMEM((DIM, CPAD), jnp.float32),
            pltpu.VMEM((1, CPAD), jnp.float32),
            pltpu.VMEM((1, CPAD), jnp.float32),
            pltpu.VMEM((1, N), jnp.float32),
        ],
    )(x, yrow, pcol, pself)
    return out[0, 0]


# speedup vs baseline: 1.6348x; 1.0386x over previous
import jax
import jax.numpy as jnp
from jax.experimental import pallas as pl
from jax.experimental.pallas import tpu as pltpu

N = 4096
DIM = 256
CLASSES = 100
CPAD = 128
STRIP = 2048
NSTRIPS = N // STRIP
SIGMA = 1.0
BIG = 3.0e38
BIGI = 2 ** 30
PACK = 128


def _dgmm_kernel(x_ref, yrow_ref, ycol_ref, out_ref,
                 mut_ref, mm_ref, cnt_ref, xxc_ref, pcol_ref):
    step = pl.program_id(0)

    @pl.when(step == 0)
    def _prologue():
        yrow = yrow_ref[...]
        cls_iota = jax.lax.broadcasted_iota(jnp.int32, (1, CPAD), 1)
        onehot = (yrow == cls_iota).astype(jnp.float32)
        counts = jnp.sum(onehot, axis=0, keepdims=True)
        x = x_ref[...]
        sums_t = jax.lax.dot_general(
            x, onehot, (((0,), (0,)), ((), ())),
            preferred_element_type=jnp.float32)
        mut = sums_t / jnp.maximum(counts, 1.0)
        mut_ref[...] = mut
        mm_ref[...] = jnp.sum(mut * mut, axis=0, keepdims=True)
        cnt_ref[...] = counts
        xsq = x * x
        ones_row = jnp.ones((1, DIM), jnp.float32)
        xxc_ref[...] = jax.lax.dot_general(
            ones_row, xsq, (((1,), (1,)), ((), ())),
            preferred_element_type=jnp.float32)
        col_iota = jax.lax.broadcasted_iota(jnp.int32, (1, N), 1)
        pcol_ref[...] = (col_iota * PACK + ycol_ref[...]).astype(jnp.float32)
        out_ref[...] = jnp.zeros_like(out_ref)

    r0 = step * STRIP
    xs = x_ref[pl.ds(r0, STRIP), :]
    yr = yrow_ref[pl.ds(r0, STRIP), :]

    xx_row = jnp.sum(xs * xs, axis=1, keepdims=True)
    xmu = jax.lax.dot_general(xs, mut_ref[...], (((1,), (0,)), ((), ())),
                              preferred_element_type=jnp.float32)
    d2 = xx_row - 2.0 * xmu + mm_ref[...]
    pi = jnp.exp(d2 * (-0.5 / (SIGMA * SIGMA)))
    pi = jnp.where(cnt_ref[...] > 0.0, pi, 0.0)
    s = jnp.sum(pi, axis=1, keepdims=True)
    pi = pi / (s + 1e-15)
    pi = jnp.clip(pi, 0.0, 1.0)
    cls_iota = jax.lax.broadcasted_iota(jnp.int32, (1, CPAD), 1)
    yh = (yr == cls_iota).astype(jnp.float32)
    gm = jnp.sum((pi - yh) ** 2, keepdims=True)

    prod = jax.lax.dot_general(xs + xs, x_ref[...], (((1,), (1,)), ((), ())),
                               preferred_element_type=jnp.float32)
    t = xxc_ref[...] - prod
    pcol = pcol_ref[...]
    row_iota = r0 + jax.lax.broadcasted_iota(jnp.int32, (STRIP, 1), 0)
    pself = (row_iota * PACK + yr).astype(jnp.float32)
    t = jnp.where(pcol == pself, BIG, t)

    labs = []
    for r in range(3):
        m = jnp.min(t, axis=1, keepdims=True)
        at_min = t == m
        pkf = jnp.min(jnp.where(at_min, pcol, BIG),
                      axis=1, keepdims=True)
        pk = pkf.astype(jnp.int32)
        labs.append(pk & (PACK - 1))
        if r < 2:
            t = jnp.where(at_min, BIG, t)

    l1, l2, l3 = labs
    a = jnp.minimum(l1, jnp.minimum(l2, l3))
    c = jnp.maximum(l1, jnp.maximum(l2, l3))
    b = l1 + l2 + l3 - a - c
    mode = jnp.where(b == c, b, a)
    diff = (mode - yr).astype(jnp.float32)
    knn = jnp.sum(diff * diff, keepdims=True)

    out_ref[...] += (gm + 0.01 * knn) * (1.0 / N)


@jax.jit
def kernel(x, y):
    yrow = y.reshape(N, 1)
    ycol = y.reshape(1, N)
    out = pl.pallas_call(
        _dgmm_kernel,
        grid=(NSTRIPS,),
        out_shape=jax.ShapeDtypeStruct((1, 1), jnp.float32),
        scratch_shapes=[
            pltpu.VMEM((DIM, CPAD), jnp.float32),
            pltpu.VMEM((1, CPAD), jnp.float32),
            pltpu.VMEM((1, CPAD), jnp.float32),
            pltpu.VMEM((1, N), jnp.float32),
            pltpu.VMEM((1, N), jnp.float32),
        ],
    )(x, yrow, ycol)
    return out[0, 0]


# speedup vs baseline: 1.6363x; 1.0009x over previous
import jax
import jax.numpy as jnp
from jax.experimental import pallas as pl
from jax.experimental.pallas import tpu as pltpu

N = 4096
DIM = 256
CLASSES = 100
CPAD = 128
STRIP = 2048
NSTRIPS = N // STRIP
SIGMA = 1.0
BIG = 3.0e38
PACK = 128


def _dgmm_kernel(x_ref, yrow_ref, ycol_ref, out_ref,
                 mut_ref, mm_ref, cnt_ref, xxc_ref, pcol_ref):
    step = pl.program_id(0)

    @pl.when(step == 0)
    def _prologue():
        yrow = yrow_ref[...]
        cls_iota = jax.lax.broadcasted_iota(jnp.int32, (1, CPAD), 1)
        onehot = (yrow == cls_iota).astype(jnp.float32)
        counts = jnp.sum(onehot, axis=0, keepdims=True)
        x = x_ref[...]
        sums_t = jax.lax.dot_general(
            x, onehot, (((0,), (0,)), ((), ())),
            preferred_element_type=jnp.float32)
        mut = sums_t / jnp.maximum(counts, 1.0)
        mut_ref[...] = mut
        mm_ref[...] = jnp.sum(mut * mut, axis=0, keepdims=True)
        cnt_ref[...] = counts
        xsq = x * x
        ones_row = jnp.ones((1, DIM), jnp.float32)
        xxc_ref[...] = jax.lax.dot_general(
            ones_row, xsq, (((1,), (1,)), ((), ())),
            preferred_element_type=jnp.float32)
        col_iota = jax.lax.broadcasted_iota(jnp.int32, (1, N), 1)
        pcol_ref[...] = (col_iota * PACK + ycol_ref[...]).astype(jnp.float32)
        out_ref[...] = jnp.zeros_like(out_ref)

    r0 = step * STRIP
    xs = x_ref[pl.ds(r0, STRIP), :]
    yr = yrow_ref[pl.ds(r0, STRIP), :]

    xx_row = jnp.sum(xs * xs, axis=1, keepdims=True)
    xmu = jax.lax.dot_general(xs, mut_ref[...], (((1,), (0,)), ((), ())),
                              preferred_element_type=jnp.float32)
    d2 = xx_row - 2.0 * xmu + mm_ref[...]
    pi = jnp.exp(d2 * (-0.5 / (SIGMA * SIGMA)))
    pi = jnp.where(cnt_ref[...] > 0.0, pi, 0.0)
    s = jnp.sum(pi, axis=1, keepdims=True)
    pi = pi / (s + 1e-15)
    pi = jnp.clip(pi, 0.0, 1.0)
    cls_iota = jax.lax.broadcasted_iota(jnp.int32, (1, CPAD), 1)
    yh = (yr == cls_iota).astype(jnp.float32)
    gm = jnp.sum((pi - yh) ** 2, keepdims=True)

    prod = jax.lax.dot_general(xs + xs, x_ref[...], (((1,), (1,)), ((), ())),
                               preferred_element_type=jnp.float32)
    t = xxc_ref[...] - prod
    pcol = pcol_ref[...]
    row_iota = r0 + jax.lax.broadcasted_iota(jnp.int32, (STRIP, 1), 0)
    pself = (row_iota * PACK + yr).astype(jnp.float32)
    t = jnp.where(pcol == pself, BIG, t)

    labs = []
    for r in range(3):
        m = jnp.min(t, axis=1, keepdims=True)
        at_min = t == m
        pkf = jnp.min(jnp.where(at_min, pcol, BIG),
                      axis=1, keepdims=True)
        pk = pkf.astype(jnp.int32)
        labs.append(pk & (PACK - 1))
        if r < 2:
            t = jnp.where(at_min, BIG, t)

    l1, l2, l3 = labs
    a = jnp.minimum(l1, jnp.minimum(l2, l3))
    c = jnp.maximum(l1, jnp.maximum(l2, l3))
    b = l1 + l2 + l3 - a - c
    mode = jnp.where(b == c, b, a)
    diff = (mode - yr).astype(jnp.float32)
    knn = jnp.sum(diff * diff, keepdims=True)

    out_ref[...] += (gm + 0.01 * knn) * (1.0 / N)


@jax.jit
def kernel(x, y):
    yrow = y.reshape(N, 1)
    ycol = y.reshape(1, N)
    out = pl.pallas_call(
        _dgmm_kernel,
        grid=(NSTRIPS,),
        out_shape=jax.ShapeDtypeStruct((1, 1), jnp.float32),
        scratch_shapes=[
            pltpu.VMEM((DIM, CPAD), jnp.float32),
            pltpu.VMEM((1, CPAD), jnp.float32),
            pltpu.VMEM((1, CPAD), jnp.float32),
            pltpu.VMEM((1, N), jnp.float32),
            pltpu.VMEM((1, N), jnp.float32),
        ],
    )(x, yrow, ycol)
    return out[0, 0]
